# Initial kernel scaffold; baseline (speedup 1.0000x reference)
import jax
import jax.numpy as jnp
from jax.experimental import pallas as pl


def kernel(q, kv, gate_W, gate_gamma, gate_beta, gate_mean, gate_var, proj_W, proj_gamma, proj_beta, proj_mean, proj_var):
    raise NotImplementedError("write your pallas kernel here")


# speedup vs baseline: 16.8476x; 16.8476x over previous
import functools

import jax
import jax.numpy as jnp
from jax.experimental import pallas as pl
from jax.experimental.pallas import tpu as pltpu

TAU = 2.0
V_TH = 1.0
BN_EPS = 1e-5

_PREC = jax.lax.Precision.HIGHEST


def _lif_unrolled(hs):
    v = jnp.zeros_like(hs[0])
    spikes = []
    for h in hs:
        v = v + (h - v) / TAU
        s = (v >= V_TH).astype(hs[0].dtype)
        v = (1.0 - s) * v
        spikes.append(s)
    return spikes


def _gate_body(kv_ref, w_ref, gamma_ref, beta_ref, mean_ref, var_ref, g_ref):
    T = kv_ref.shape[0]
    scale = gamma_ref[...] * jax.lax.rsqrt(var_ref[...] + BN_EPS)
    shift = beta_ref[...] - mean_ref[...] * scale
    hs = []
    for t in range(T):
        h = jax.lax.dot_general(kv_ref[t], w_ref[...], (((1,), (1,)), ((), ())),
                                preferred_element_type=jnp.float32,
                                precision=_PREC)
        hs.append(h * scale + shift)
    spikes = _lif_unrolled(hs)
    g_ref[...] = sum(spikes) / float(T)


def _main_body(q_ref, g_ref, w_ref, gamma_ref, beta_ref, mean_ref, var_ref,
               out_ref, *, nkv):
    T, blk, D = q_ref.shape
    sscale = float(D) ** (-0.5)
    bscale = gamma_ref[...] * jax.lax.rsqrt(var_ref[...] + BN_EPS)
    bshift = beta_ref[...] - mean_ref[...] * bscale
    col = jax.lax.broadcasted_iota(jnp.int32, (blk, nkv), 1)
    neg = jnp.float32(-3.4e38)
    hs = []
    for t in range(T):
        s = jax.lax.dot_general(q_ref[t], g_ref[...], (((1,), (1,)), ((), ())),
                                preferred_element_type=jnp.float32,
                                precision=_PREC) * sscale
        m1 = jnp.max(s, axis=1, keepdims=True)
        i1 = jnp.min(jnp.where(s == m1, col, nkv), axis=1, keepdims=True)
        s2 = jnp.where(col == i1, neg, s)
        m2 = jnp.max(s2, axis=1, keepdims=True)
        i2 = jnp.min(jnp.where(s2 == m2, col, nkv), axis=1, keepdims=True)
        w = jnp.where(col == i1, m1, 0.0) + jnp.where(col == i2, m2, 0.0)
        upd = jax.lax.dot_general(w, g_ref[...], (((1,), (0,)), ((), ())),
                                  preferred_element_type=jnp.float32,
                                  precision=_PREC)
        h = jax.lax.dot_general(upd, w_ref[...], (((1,), (1,)), ((), ())),
                                preferred_element_type=jnp.float32,
                                precision=_PREC)
        hs.append(h * bscale + bshift)
    spikes = _lif_unrolled(hs)
    for t in range(T):
        out_ref[t] = spikes[t]


def kernel(q, kv, gate_W, gate_gamma, gate_beta, gate_mean, gate_var,
           proj_W, proj_gamma, proj_beta, proj_mean, proj_var, *,
           interpret=False):
    T, B, Nq, D = q.shape
    Nkv = kv.shape[2]
    kv3 = kv.reshape(T, B * Nkv, D)
    q3 = q.reshape(T, B * Nq, D)
    row = lambda a: a.reshape(1, D)

    blk_g = 512
    g = pl.pallas_call(
        _gate_body,
        grid=(Nkv // blk_g,),
        in_specs=[
            pl.BlockSpec((T, blk_g, D), lambda i: (0, i, 0)),
            pl.BlockSpec((D, D), lambda i: (0, 0)),
            pl.BlockSpec((1, D), lambda i: (0, 0)),
            pl.BlockSpec((1, D), lambda i: (0, 0)),
            pl.BlockSpec((1, D), lambda i: (0, 0)),
            pl.BlockSpec((1, D), lambda i: (0, 0)),
        ],
        out_specs=pl.BlockSpec((blk_g, D), lambda i: (i, 0)),
        out_shape=jax.ShapeDtypeStruct((Nkv, D), jnp.float32),
        interpret=interpret,
    )(kv3, gate_W, row(gate_gamma), row(gate_beta), row(gate_mean),
      row(gate_var))

    blk_q = 256
    out = pl.pallas_call(
        functools.partial(_main_body, nkv=Nkv),
        grid=(Nq // blk_q,),
        in_specs=[
            pl.BlockSpec((T, blk_q, D), lambda i: (0, i, 0)),
            pl.BlockSpec((Nkv, D), lambda i: (0, 0)),
            pl.BlockSpec((D, D), lambda i: (0, 0)),
            pl.BlockSpec((1, D), lambda i: (0, 0)),
            pl.BlockSpec((1, D), lambda i: (0, 0)),
            pl.BlockSpec((1, D), lambda i: (0, 0)),
            pl.BlockSpec((1, D), lambda i: (0, 0)),
        ],
        out_specs=pl.BlockSpec((T, blk_q, D), lambda i: (0, i, 0)),
        out_shape=jax.ShapeDtypeStruct((T, Nq, D), jnp.float32),
        interpret=interpret,
    )(q3, g, proj_W, row(proj_gamma), row(proj_beta), row(proj_mean),
      row(proj_var))

    return out.reshape(T, B, Nq, D)


# speedup vs baseline: 45.4754x; 2.6992x over previous
import functools

import jax
import jax.numpy as jnp
from jax.experimental import pallas as pl
from jax.experimental.pallas import tpu as pltpu

TAU = 2.0
V_TH = 1.0
BN_EPS = 1e-5

_PREC = jax.lax.Precision.HIGHEST


def _lif_unrolled(hs):
    v = jnp.zeros_like(hs[0])
    spikes = []
    for h in hs:
        v = v + (h - v) / TAU
        s = (v >= V_TH).astype(hs[0].dtype)
        v = (1.0 - s) * v
        spikes.append(s)
    return spikes


def _gate_body(kv_ref, w_ref, gamma_ref, beta_ref, mean_ref, var_ref, g_ref):
    T = kv_ref.shape[0]
    scale = gamma_ref[...] * jax.lax.rsqrt(var_ref[...] + BN_EPS)
    shift = beta_ref[...] - mean_ref[...] * scale
    hs = []
    for t in range(T):
        h = jax.lax.dot_general(kv_ref[t], w_ref[...], (((1,), (1,)), ((), ())),
                                preferred_element_type=jnp.float32,
                                precision=_PREC)
        hs.append(h * scale + shift)
    spikes = _lif_unrolled(hs)
    g_ref[...] = sum(spikes) / float(T)


def _main_body(q_ref, g_ref, w_ref, gamma_ref, beta_ref, mean_ref, var_ref,
               out_ref, *, nkv):
    T, blk, D = q_ref.shape
    sscale = float(D) ** (-0.5)
    bscale = gamma_ref[...] * jax.lax.rsqrt(var_ref[...] + BN_EPS)
    bshift = beta_ref[...] - mean_ref[...] * bscale
    col = jax.lax.broadcasted_iota(jnp.int32, (blk, nkv), 1)
    neg = jnp.float32(-3.4e38)
    hs = []
    for t in range(T):
        s = jax.lax.dot_general(q_ref[t], g_ref[...], (((1,), (1,)), ((), ())),
                                preferred_element_type=jnp.float32,
                                precision=_PREC) * sscale
        m1 = jnp.max(s, axis=1, keepdims=True)
        i1 = jnp.min(jnp.where(s == m1, col, nkv), axis=1, keepdims=True)
        s2 = jnp.where(col == i1, neg, s)
        m2 = jnp.max(s2, axis=1, keepdims=True)
        i2 = jnp.min(jnp.where(s2 == m2, col, nkv), axis=1, keepdims=True)
        oh1 = jnp.where(col == i1, 1.0, 0.0).astype(jnp.bfloat16)
        oh2 = jnp.where(col == i2, 1.0, 0.0).astype(jnp.bfloat16)
        gb = g_ref[...].astype(jnp.bfloat16)
        g1 = jax.lax.dot_general(oh1, gb, (((1,), (0,)), ((), ())),
                                 preferred_element_type=jnp.float32)
        g2 = jax.lax.dot_general(oh2, gb, (((1,), (0,)), ((), ())),
                                 preferred_element_type=jnp.float32)
        upd = m1 * g1 + m2 * g2
        h = jax.lax.dot_general(upd, w_ref[...], (((1,), (1,)), ((), ())),
                                preferred_element_type=jnp.float32,
                                precision=_PREC)
        hs.append(h * bscale + bshift)
    spikes = _lif_unrolled(hs)
    for t in range(T):
        out_ref[t] = spikes[t]


def kernel(q, kv, gate_W, gate_gamma, gate_beta, gate_mean, gate_var,
           proj_W, proj_gamma, proj_beta, proj_mean, proj_var, *,
           interpret=False):
    T, B, Nq, D = q.shape
    Nkv = kv.shape[2]
    kv3 = kv.reshape(T, B * Nkv, D)
    q3 = q.reshape(T, B * Nq, D)
    row = lambda a: a.reshape(1, D)

    blk_g = 512
    g = pl.pallas_call(
        _gate_body,
        grid=(Nkv // blk_g,),
        in_specs=[
            pl.BlockSpec((T, blk_g, D), lambda i: (0, i, 0)),
            pl.BlockSpec((D, D), lambda i: (0, 0)),
            pl.BlockSpec((1, D), lambda i: (0, 0)),
            pl.BlockSpec((1, D), lambda i: (0, 0)),
            pl.BlockSpec((1, D), lambda i: (0, 0)),
            pl.BlockSpec((1, D), lambda i: (0, 0)),
        ],
        out_specs=pl.BlockSpec((blk_g, D), lambda i: (i, 0)),
        out_shape=jax.ShapeDtypeStruct((Nkv, D), jnp.float32),
        interpret=interpret,
    )(kv3, gate_W, row(gate_gamma), row(gate_beta), row(gate_mean),
      row(gate_var))

    blk_q = 256
    out = pl.pallas_call(
        functools.partial(_main_body, nkv=Nkv),
        grid=(Nq // blk_q,),
        in_specs=[
            pl.BlockSpec((T, blk_q, D), lambda i: (0, i, 0)),
            pl.BlockSpec((Nkv, D), lambda i: (0, 0)),
            pl.BlockSpec((D, D), lambda i: (0, 0)),
            pl.BlockSpec((1, D), lambda i: (0, 0)),
            pl.BlockSpec((1, D), lambda i: (0, 0)),
            pl.BlockSpec((1, D), lambda i: (0, 0)),
            pl.BlockSpec((1, D), lambda i: (0, 0)),
        ],
        out_specs=pl.BlockSpec((T, blk_q, D), lambda i: (0, i, 0)),
        out_shape=jax.ShapeDtypeStruct((T, Nq, D), jnp.float32),
        interpret=interpret,
    )(q3, g, proj_W, row(proj_gamma), row(proj_beta), row(proj_mean),
      row(proj_var))

    return out.reshape(T, B, Nq, D)
